# baseline (speedup 1.0000x reference)
import functools

import jax
import jax.numpy as jnp
from jax import lax
from jax.experimental import pallas as pl
from jax.experimental.pallas import tpu as pltpu
from jax.experimental.pallas import tpu_sc as plsc

DIM = 128
NC = 2
NS = 16
NW = NC * NS
CHUNK = 128
NBUF = 3
DEPTH = 2


def _sc_gather_body(pos_hbm, cos_hbm, sin_hbm, cos_out, sin_out,
                    idx_v, cbufs, sbufs, isem, gsems, osems,
                    n_chunks, w_per_row):
    wid = lax.axis_index("s") * NC + lax.axis_index("c")
    rows_per_w = n_chunks * CHUNK
    b = wid // w_per_row
    off = (wid % w_per_row) * rows_per_w
    pltpu.async_copy(pos_hbm.at[b, pl.ds(off, rows_per_w)], idx_v, isem).wait()

    gathers = {}
    outs = {}
    for j in range(n_chunks + DEPTH):
        if j < n_chunks:
            bu = j % NBUF
            if j >= NBUF:
                outs[j - NBUF][0].wait()
                outs[j - NBUF][1].wait()
            idx_c = idx_v.at[pl.ds(j * CHUNK, CHUNK)]
            gathers[j] = (
                pltpu.async_copy(cos_hbm.at[idx_c], cbufs[bu], gsems[2 * bu]),
                pltpu.async_copy(sin_hbm.at[idx_c], sbufs[bu], gsems[2 * bu + 1]),
            )
        if j >= DEPTH:
            jj = j - DEPTH
            bu = jj % NBUF
            gathers[jj][0].wait()
            gathers[jj][1].wait()
            row0 = wid * rows_per_w + jj * CHUNK
            outs[jj] = (
                pltpu.async_copy(cbufs[bu], cos_out.at[pl.ds(row0, CHUNK)], osems[2 * bu]),
                pltpu.async_copy(sbufs[bu], sin_out.at[pl.ds(row0, CHUNK)], osems[2 * bu + 1]),
            )
    for jj in range(max(n_chunks - NBUF, 0), n_chunks):
        outs[jj][0].wait()
        outs[jj][1].wait()


@jax.jit
def _rope_gather(positions, cos_cached, sin_cached):
    batch, seq = positions.shape
    total = batch * seq
    n_chunks = total // (NW * CHUNK)
    rows_per_w = n_chunks * CHUNK
    w_per_row = seq // rows_per_w
    mesh = plsc.VectorSubcoreMesh(core_axis_name="c", subcore_axis_name="s")
    scratch = (
        pltpu.VMEM((rows_per_w,), jnp.int32),
        [pltpu.VMEM((CHUNK, DIM), jnp.float32) for _ in range(NBUF)],
        [pltpu.VMEM((CHUNK, DIM), jnp.float32) for _ in range(NBUF)],
        pltpu.SemaphoreType.DMA,
        [pltpu.SemaphoreType.DMA for _ in range(2 * NBUF)],
        [pltpu.SemaphoreType.DMA for _ in range(2 * NBUF)],
    )
    out_type = (
        jax.ShapeDtypeStruct((total, DIM), jnp.float32),
        jax.ShapeDtypeStruct((total, DIM), jnp.float32),
    )
    body = functools.partial(_sc_gather_body, n_chunks=n_chunks,
                             w_per_row=w_per_row)
    return pl.kernel(
        body,
        out_type=out_type,
        mesh=mesh,
        scratch_types=scratch,
    )(positions, cos_cached, sin_cached)


def kernel(positions, cos_cached, sin_cached):
    batch, seq = positions.shape
    cos_flat, sin_flat = _rope_gather(positions, cos_cached, sin_cached)
    return (cos_flat.reshape(batch, seq, DIM), sin_flat.reshape(batch, seq, DIM))
